# baseline (speedup 1.0000x reference)
import jax
import jax.numpy as jnp
from jax import lax
from jax.experimental import pallas as pl
from jax.experimental.pallas import tpu as pltpu

B, T, D = 16, 4096, 300
BT = 512


def _tc_body2(x_ref, o_ref):
    o_ref[...] = x_ref[...]


def _tc_body(xlen_ref, x_ref, o_ref):
    b = pl.program_id(0)
    tb = pl.program_id(1)
    xlen = xlen_ref[b]
    rows = tb * BT + lax.broadcasted_iota(jnp.int32, (1, BT, 1), 1)
    o_ref[...] = jnp.where(rows < xlen, x_ref[...], 0.0)


def _masked_copy_tc(x, x_len):
    return pl.pallas_call(
        _tc_body2,
        grid=(B, T // BT),
        in_specs=[pl.BlockSpec((1, BT, D), lambda b, tb: (b, tb, 0))],
        out_specs=pl.BlockSpec((1, BT, D), lambda b, tb: (b, tb, 0)),
        out_shape=jax.ShapeDtypeStruct((B, T, D), jnp.float32),
    )(x)


def kernel(x, x_len):
    return _masked_copy_tc(x, x_len.astype(jnp.int32))


# speedup vs baseline: 1.2101x; 1.2101x over previous
import jax
import jax.numpy as jnp
from jax import lax
from jax.experimental import pallas as pl
from jax.experimental.pallas import tpu as pltpu

B, T, D = 16, 4096, 300
BT = 4096


def _tc_body2(x_ref, o_ref):
    o_ref[...] = x_ref[...]


def _tc_body(xlen_ref, x_ref, o_ref):
    b = pl.program_id(0)
    tb = pl.program_id(1)
    xlen = xlen_ref[b]
    rows = tb * BT + lax.broadcasted_iota(jnp.int32, (1, BT, 1), 1)
    o_ref[...] = jnp.where(rows < xlen, x_ref[...], 0.0)


def _masked_copy_tc(x, x_len):
    return pl.pallas_call(
        _tc_body2,
        grid=(B, T // BT),
        in_specs=[pl.BlockSpec((1, BT, D), lambda b, tb: (b, tb, 0))],
        out_specs=pl.BlockSpec((1, BT, D), lambda b, tb: (b, tb, 0)),
        out_shape=jax.ShapeDtypeStruct((B, T, D), jnp.float32),
    )(x)


def kernel(x, x_len):
    return _masked_copy_tc(x, x_len.astype(jnp.int32))


# speedup vs baseline: 5.5367x; 4.5753x over previous
import jax
import jax.numpy as jnp
from jax import lax
from jax.experimental import pallas as pl
from jax.experimental.pallas import tpu as pltpu

B, T, D = 16, 4096, 300
DBLK = 25


def _tc_body(x_ref, xl_ref, o_ref):
    xl = xl_ref[...][:, 0:1]
    tio = lax.broadcasted_iota(jnp.int32, (B, T), 1)
    mask = tio < xl
    o_ref[...] = jnp.where(mask[None, :, :], x_ref[...], 0.0)


def _masked_copy_tc(xt, xl2d):
    return pl.pallas_call(
        _tc_body,
        grid=(D // DBLK,),
        in_specs=[
            pl.BlockSpec((DBLK, B, T), lambda i: (i, 0, 0)),
            pl.BlockSpec((B, 128), lambda i: (0, 0)),
        ],
        out_specs=pl.BlockSpec((DBLK, B, T), lambda i: (i, 0, 0)),
        out_shape=jax.ShapeDtypeStruct((D, B, T), jnp.float32),
    )(xt, xl2d)


def kernel(x, x_len):
    xt = lax.transpose(x, (2, 0, 1))
    xl2d = jnp.broadcast_to(x_len.astype(jnp.int32)[:, None], (B, 128))
    out_t = _masked_copy_tc(xt, xl2d)
    return lax.transpose(out_t, (1, 2, 0))


# speedup vs baseline: 5.6838x; 1.0266x over previous
import jax
import jax.numpy as jnp
from jax import lax
from jax.experimental import pallas as pl
from jax.experimental.pallas import tpu as pltpu

B, T, D = 16, 4096, 300
DBLK = 50


def _tc_body(x_ref, xl_ref, o_ref):
    xl = xl_ref[...][:, 0:1]
    tio = lax.broadcasted_iota(jnp.int32, (B, T), 1)
    mask = tio < xl
    o_ref[...] = jnp.where(mask[None, :, :], x_ref[...], 0.0)


def _masked_copy_tc(xt, xl2d):
    return pl.pallas_call(
        _tc_body,
        grid=(D // DBLK,),
        in_specs=[
            pl.BlockSpec((DBLK, B, T), lambda i: (i, 0, 0)),
            pl.BlockSpec((B, 128), lambda i: (0, 0)),
        ],
        out_specs=pl.BlockSpec((DBLK, B, T), lambda i: (i, 0, 0)),
        out_shape=jax.ShapeDtypeStruct((D, B, T), jnp.float32),
    )(xt, xl2d)


def kernel(x, x_len):
    xt = lax.transpose(x, (2, 0, 1))
    xl2d = jnp.broadcast_to(x_len.astype(jnp.int32)[:, None], (B, 128))
    out_t = _masked_copy_tc(xt, xl2d)
    return lax.transpose(out_t, (1, 2, 0))
